# Initial kernel scaffold; baseline (speedup 1.0000x reference)
import jax
import jax.numpy as jnp
from jax import lax
from jax.experimental import pallas as pl
from jax.experimental.pallas import tpu as pltpu
from jax.experimental.pallas import tpu_sc as plsc

N = 10000
E = 320000
D = 128
NCLS = 40

NC = 2
NS = 16
NW = NC * NS
EPW = E // NW
CH = 80
NCHUNK = EPW // CH
RPT = N // NS
ZR = 125

_mesh = plsc.VectorSubcoreMesh(core_axis_name="c", subcore_axis_name="s",
                               num_cores=NC, num_subcores=NS)


def _deg_body(cols_hbm, out_hbm, cols_v, ones_v, zbuf, dacc):
    cid = lax.axis_index("c")
    sid = lax.axis_index("s")
    wid = cid * NS + sid

    pltpu.sync_copy(cols_hbm.at[wid], cols_v)

    lane = lax.iota(jnp.int32, 16)
    one0 = jnp.where(lane == 0, 1.0, 0.0).astype(jnp.float32)
    zero = jnp.zeros((16,), jnp.float32)

    @pl.loop(0, CH)
    def _(i):
        ones_v[i, :] = one0

    @pl.loop(0, ZR)
    def _(i):
        zbuf[i, :] = zero

    base = sid * RPT
    for k in range(RPT // ZR):
        pltpu.sync_copy(zbuf, dacc.at[pl.ds(base + k * ZR, ZR)])
    plsc.subcore_barrier()

    @pl.loop(0, NCHUNK)
    def _(j):
        pltpu.sync_copy(ones_v, dacc.at[cols_v.at[j]], add=True)

    plsc.subcore_barrier()
    pltpu.sync_copy(dacc.at[pl.ds(base, RPT)],
                    out_hbm.at[cid, pl.ds(base, RPT)])


_deg_kernel = pl.kernel(
    _deg_body,
    out_type=jax.ShapeDtypeStruct((NC, N, 16), jnp.float32),
    mesh=_mesh,
    scratch_types=[
        pltpu.VMEM((NCHUNK, CH), jnp.int32),
        pltpu.VMEM((CH, 16), jnp.float32),
        pltpu.VMEM((ZR, 16), jnp.float32),
        pltpu.VMEM_SHARED((N, 16), jnp.float32),
    ],
)


def _hop_body(y_hbm, rows_hbm, cols_hbm, out_hbm,
              rows_v, cols_v, buf, zbuf, acc):
    cid = lax.axis_index("c")
    sid = lax.axis_index("s")
    wid = cid * NS + sid

    pltpu.sync_copy(rows_hbm.at[wid], rows_v)
    pltpu.sync_copy(cols_hbm.at[wid], cols_v)

    zero = jnp.zeros((16,), jnp.float32)

    @pl.loop(0, ZR)
    def _(i):
        for c in range(D // 16):
            zbuf[i, pl.ds(c * 16, 16)] = zero

    base = sid * RPT
    for k in range(RPT // ZR):
        pltpu.sync_copy(zbuf, acc.at[pl.ds(base + k * ZR, ZR)])
    plsc.subcore_barrier()

    @pl.loop(0, NCHUNK)
    def _(j):
        pltpu.sync_copy(y_hbm.at[rows_v.at[j]], buf)
        pltpu.sync_copy(buf, acc.at[cols_v.at[j]], add=True)

    plsc.subcore_barrier()
    pltpu.sync_copy(acc.at[pl.ds(base, RPT)],
                    out_hbm.at[cid, pl.ds(base, RPT)])


_hop_kernel = pl.kernel(
    _hop_body,
    out_type=jax.ShapeDtypeStruct((NC, N, D), jnp.float32),
    mesh=_mesh,
    scratch_types=[
        pltpu.VMEM((NCHUNK, CH), jnp.int32),
        pltpu.VMEM((NCHUNK, CH), jnp.int32),
        pltpu.VMEM((CH, D), jnp.float32),
        pltpu.VMEM((ZR, D), jnp.float32),
        pltpu.VMEM_SHARED((N, D), jnp.float32),
    ],
)


def _prep_body(degp_ref, x_ref, dinv_ref, y0_ref):
    deg = 1.0 + degp_ref[0, :, 0:1] + degp_ref[1, :, 0:1]
    dv = lax.rsqrt(deg)
    dinv_ref[...] = dv
    y0_ref[...] = x_ref[...] * dv


def _prep(degp, x):
    return pl.pallas_call(
        _prep_body,
        out_shape=[jax.ShapeDtypeStruct((N, 1), jnp.float32),
                   jax.ShapeDtypeStruct((N, D), jnp.float32)],
    )(degp, x)


def _mid_body(a_ref, y0_ref, dinv_ref, y1_ref):
    dv = dinv_ref[...]
    y1_ref[...] = (a_ref[0] + a_ref[1] + y0_ref[...]) * (dv * dv)


def _mid(a, y0, dinv2):
    return pl.pallas_call(
        _mid_body,
        out_shape=jax.ShapeDtypeStruct((N, D), jnp.float32),
    )(a, y0, dinv2)


_RB = 1000


def _final_body(b_ref, y1_ref, dinv_ref, wgc_ref, bgc_ref, wfc_ref, bfc_ref,
                o_ref):
    dv = dinv_ref[...]
    h2 = (b_ref[0] + b_ref[1] + y1_ref[...]) * dv
    t = jnp.dot(h2, wgc_ref[...], preferred_element_type=jnp.float32)
    t = jnp.maximum(t + bgc_ref[...], 0.0)
    t2 = jnp.dot(t, wfc_ref[...], preferred_element_type=jnp.float32)
    t2 = t2 + bfc_ref[...]
    m = jnp.max(t2, axis=1, keepdims=True)
    e = jnp.exp(t2 - m)
    s = jnp.sum(e, axis=1, keepdims=True)
    o_ref[...] = t2 - m - jnp.log(s)


def _final(b, y1, dinv2, W_gc, b_gc, W_fc, b_fc):
    return pl.pallas_call(
        _final_body,
        grid=(N // _RB,),
        in_specs=[
            pl.BlockSpec((NC, _RB, D), lambda i: (0, i, 0)),
            pl.BlockSpec((_RB, D), lambda i: (i, 0)),
            pl.BlockSpec((_RB, 1), lambda i: (i, 0)),
            pl.BlockSpec((D, D), lambda i: (0, 0)),
            pl.BlockSpec((1, D), lambda i: (0, 0)),
            pl.BlockSpec((D, NCLS), lambda i: (0, 0)),
            pl.BlockSpec((1, NCLS), lambda i: (0, 0)),
        ],
        out_specs=pl.BlockSpec((_RB, NCLS), lambda i: (i, 0)),
        out_shape=jax.ShapeDtypeStruct((N, NCLS), jnp.float32),
    )(b, y1, dinv2, W_gc, b_gc, W_fc, b_fc)


def kernel(x, edge_index, W_gc, b_gc, W_fc, b_fc):
    ei = edge_index.astype(jnp.int32)
    rows3 = ei[0].reshape(NW, NCHUNK, CH)
    cols3 = ei[1].reshape(NW, NCHUNK, CH)

    degp = _deg_kernel(cols3)
    dinv2, y0 = _prep(degp, x)
    a = _hop_kernel(y0, rows3, cols3)
    y1 = _mid(a, y0, dinv2)
    b = _hop_kernel(y1, rows3, cols3)
    return _final(b, y1, dinv2, W_gc, b_gc.reshape(1, D),
                  W_fc, b_fc.reshape(1, NCLS))


# speedup vs baseline: 15.7018x; 15.7018x over previous
import jax
import jax.numpy as jnp
import numpy as np
from jax import lax
from jax.experimental import pallas as pl
from jax.experimental.pallas import tpu as pltpu
from jax.experimental.pallas import tpu_sc as plsc

N = 10000
NP = 10240
E = 320000
EP = 327680
D = 128
NCLS = 40

NC = 2
NS = 16
NW = NC * NS
EPW = EP // NW
CH = 64
NCHUNK = EPW // CH
RPT = NP // NS

_mesh = plsc.VectorSubcoreMesh(core_axis_name="c", subcore_axis_name="s",
                               num_cores=NC, num_subcores=NS)


def _deg_body(cols_hbm, out_hbm, cols_v, ones_v, dacc):
    cid = lax.axis_index("c")
    sid = lax.axis_index("s")
    wid = cid * NS + sid

    pltpu.sync_copy(cols_hbm.at[wid], cols_v)

    zero = jnp.zeros((16,), jnp.float32)
    one = jnp.ones((16,), jnp.float32)

    @pl.loop(0, CH)
    def _(i):
        for c in range(D // 16):
            ones_v[i, pl.ds(c * 16, 16)] = zero

    base = sid * RPT
    for k in range(RPT // CH):
        pltpu.sync_copy(ones_v, dacc.at[pl.ds(base + k * CH, CH)])

    @pl.loop(0, CH)
    def _(i):
        for c in range(D // 16):
            ones_v[i, pl.ds(c * 16, 16)] = one

    plsc.subcore_barrier()

    @pl.loop(0, NCHUNK)
    def _(j):
        pltpu.sync_copy(ones_v, dacc.at[cols_v.at[j]], add=True)

    plsc.subcore_barrier()
    pltpu.sync_copy(dacc.at[pl.ds(base, RPT)],
                    out_hbm.at[cid, pl.ds(base, RPT)])


_deg_kernel = pl.kernel(
    _deg_body,
    out_type=jax.ShapeDtypeStruct((NC, NP, D), jnp.float32),
    mesh=_mesh,
    scratch_types=[
        pltpu.VMEM((NCHUNK, CH), jnp.int32),
        pltpu.VMEM((CH, D), jnp.float32),
        pltpu.VMEM_SHARED((NP, D), jnp.float32),
    ],
)


def _hop_body(y_hbm, rows_hbm, cols_hbm, out_hbm,
              rows_v, cols_v, buf, acc):
    cid = lax.axis_index("c")
    sid = lax.axis_index("s")
    wid = cid * NS + sid

    pltpu.sync_copy(rows_hbm.at[wid], rows_v)
    pltpu.sync_copy(cols_hbm.at[wid], cols_v)

    zero = jnp.zeros((16,), jnp.float32)

    @pl.loop(0, CH)
    def _(i):
        for c in range(D // 16):
            buf[i, pl.ds(c * 16, 16)] = zero

    base = sid * RPT
    for k in range(RPT // CH):
        pltpu.sync_copy(buf, acc.at[pl.ds(base + k * CH, CH)])
    plsc.subcore_barrier()

    @pl.loop(0, NCHUNK)
    def _(j):
        pltpu.sync_copy(y_hbm.at[rows_v.at[j]], buf)
        pltpu.sync_copy(buf, acc.at[cols_v.at[j]], add=True)

    plsc.subcore_barrier()
    pltpu.sync_copy(acc.at[pl.ds(base, RPT)],
                    out_hbm.at[cid, pl.ds(base, RPT)])


_hop_kernel = pl.kernel(
    _hop_body,
    out_type=jax.ShapeDtypeStruct((NC, NP, D), jnp.float32),
    mesh=_mesh,
    scratch_types=[
        pltpu.VMEM((NCHUNK, CH), jnp.int32),
        pltpu.VMEM((NCHUNK, CH), jnp.int32),
        pltpu.VMEM((CH, D), jnp.float32),
        pltpu.VMEM_SHARED((NP, D), jnp.float32),
    ],
)


def _prep_body(degp_ref, x_ref, dinv_ref, sdeg_ref, y0_ref):
    deg = 1.0 + degp_ref[0, :, 0:1] + degp_ref[1, :, 0:1]
    dv = lax.rsqrt(deg)
    dinv_ref[...] = dv
    sdeg_ref[...] = jnp.sqrt(deg)
    y0_ref[...] = x_ref[...] * dv


def _prep(degp, x):
    return pl.pallas_call(
        _prep_body,
        out_shape=[jax.ShapeDtypeStruct((NP, 1), jnp.float32),
                   jax.ShapeDtypeStruct((NP, 1), jnp.float32),
                   jax.ShapeDtypeStruct((NP, D), jnp.float32)],
    )(degp, x)


def _mid_body(a_ref, y0_ref, dinv_ref, y1_ref):
    dv = dinv_ref[...]
    y1_ref[...] = (a_ref[0] + a_ref[1] + y0_ref[...]) * (dv * dv)


def _mid(a, y0, dinv2):
    return pl.pallas_call(
        _mid_body,
        out_shape=jax.ShapeDtypeStruct((NP, D), jnp.float32),
    )(a, y0, dinv2)


_RB = 640


def _final_body(y2_ref, sdeg_ref, wgc_ref, bgc_ref, wfc_ref, bfc_ref,
                o_ref):
    h2 = y2_ref[...] * sdeg_ref[...]
    t = jnp.dot(h2, wgc_ref[...], preferred_element_type=jnp.float32)
    t = jnp.maximum(t + bgc_ref[...], 0.0)
    t2 = jnp.dot(t, wfc_ref[...], preferred_element_type=jnp.float32)
    t2 = t2 + bfc_ref[...]
    m = jnp.max(t2, axis=1, keepdims=True)
    e = jnp.exp(t2 - m)
    s = jnp.sum(e, axis=1, keepdims=True)
    o_ref[...] = t2 - m - jnp.log(s)


def _final(y2, sdeg, W_gc, b_gc, W_fc, b_fc):
    return pl.pallas_call(
        _final_body,
        grid=(NP // _RB,),
        in_specs=[
            pl.BlockSpec((_RB, D), lambda i: (i, 0)),
            pl.BlockSpec((_RB, 1), lambda i: (i, 0)),
            pl.BlockSpec((D, D), lambda i: (0, 0)),
            pl.BlockSpec((1, D), lambda i: (0, 0)),
            pl.BlockSpec((D, NCLS), lambda i: (0, 0)),
            pl.BlockSpec((1, NCLS), lambda i: (0, 0)),
        ],
        out_specs=pl.BlockSpec((_RB, NCLS), lambda i: (i, 0)),
        out_shape=jax.ShapeDtypeStruct((NP, NCLS), jnp.float32),
    )(y2, sdeg, W_gc, b_gc, W_fc, b_fc)


_PAD_IDX = np.arange(EP - E, dtype=np.int32) % (NP - N) + N


def kernel(x, edge_index, W_gc, b_gc, W_fc, b_fc):
    ei = edge_index.astype(jnp.int32)
    pad = jnp.asarray(_PAD_IDX)
    rows2 = jnp.concatenate([ei[0], pad]).reshape(NW, NCHUNK, CH)
    cols2 = jnp.concatenate([ei[1], pad]).reshape(NW, NCHUNK, CH)
    xp = jnp.pad(x, ((0, NP - N), (0, 0)))

    degp = _deg_kernel(cols2)
    dinv2, sdeg, y0 = _prep(degp, xp)

    def hop_step(_, y):
        a = _hop_kernel(y, rows2, cols2)
        return _mid(a, y, dinv2)

    y2 = lax.fori_loop(0, 2, hop_step, y0)
    out = _final(y2, sdeg, W_gc, b_gc.reshape(1, D),
                 W_fc, b_fc.reshape(1, NCLS))
    return out[:N]


# speedup vs baseline: 18.8126x; 1.1981x over previous
import jax
import jax.numpy as jnp
import numpy as np
from jax import lax
from jax.experimental import pallas as pl
from jax.experimental.pallas import tpu as pltpu
from jax.experimental.pallas import tpu_sc as plsc

N = 10000
NP = 10240
E = 320000
EP = 327680
D = 128
NCLS = 40

NC = 2
NS = 16
NW = NC * NS
EPW = EP // NW
CH = 64
NCHUNK = EPW // CH
RPT = NP // NS

_mesh = plsc.VectorSubcoreMesh(core_axis_name="c", subcore_axis_name="s",
                               num_cores=NC, num_subcores=NS)


def _deg_body(cols_hbm, out_hbm, cols_v, ones_v, dacc):
    cid = lax.axis_index("c")
    sid = lax.axis_index("s")
    wid = cid * NS + sid

    pltpu.sync_copy(cols_hbm.at[wid], cols_v)

    zero = jnp.zeros((16,), jnp.float32)
    one = jnp.ones((16,), jnp.float32)

    @pl.loop(0, CH)
    def _(i):
        for c in range(D // 16):
            ones_v[i, pl.ds(c * 16, 16)] = zero

    base = sid * RPT
    for k in range(RPT // CH):
        pltpu.sync_copy(ones_v, dacc.at[pl.ds(base + k * CH, CH)])

    @pl.loop(0, CH)
    def _(i):
        for c in range(D // 16):
            ones_v[i, pl.ds(c * 16, 16)] = one

    plsc.subcore_barrier()

    @pl.loop(0, NCHUNK)
    def _(j):
        pltpu.sync_copy(ones_v, dacc.at[cols_v.at[j]], add=True)

    plsc.subcore_barrier()
    pltpu.sync_copy(dacc.at[pl.ds(base, RPT)],
                    out_hbm.at[cid, pl.ds(base, RPT)])


_deg_kernel = pl.kernel(
    _deg_body,
    out_type=jax.ShapeDtypeStruct((NC, NP, D), jnp.float32),
    mesh=_mesh,
    scratch_types=[
        pltpu.VMEM((NCHUNK, CH), jnp.int32),
        pltpu.VMEM((CH, D), jnp.float32),
        pltpu.VMEM_SHARED((NP, D), jnp.float32),
    ],
)


def _hop_body(y_hbm, rows_hbm, cols_hbm, out_hbm,
              rows_v, cols_v, buf0, buf1, acc, gsem0, gsem1):
    cid = lax.axis_index("c")
    sid = lax.axis_index("s")
    wid = cid * NS + sid

    zero = jnp.zeros((16,), jnp.float32)

    @pl.loop(0, CH)
    def _(i):
        for c in range(D // 16):
            buf0[i, pl.ds(c * 16, 16)] = zero

    base = sid * RPT
    for k in range(RPT // CH):
        pltpu.sync_copy(buf0, acc.at[pl.ds(base + k * CH, CH)])
    plsc.subcore_barrier()

    HC = NCHUNK // 2
    for h in range(2):
        pltpu.sync_copy(rows_hbm.at[wid, pl.ds(h * HC, HC)], rows_v)
        pltpu.sync_copy(cols_hbm.at[wid, pl.ds(h * HC, HC)], cols_v)
        pltpu.async_copy(y_hbm.at[rows_v.at[0]], buf0, gsem0)

        @pl.loop(0, HC // 2)
        def _(g):
            j0 = 2 * g
            j1 = j0 + 1
            j2 = jnp.minimum(j0 + 2, HC - 1)
            pltpu.make_async_copy(y_hbm.at[rows_v.at[j0]], buf0,
                                  gsem0).wait()
            pltpu.async_copy(y_hbm.at[rows_v.at[j1]], buf1, gsem1)
            pltpu.sync_copy(buf0, acc.at[cols_v.at[j0]], add=True)
            pltpu.make_async_copy(y_hbm.at[rows_v.at[j1]], buf1,
                                  gsem1).wait()
            pltpu.async_copy(y_hbm.at[rows_v.at[j2]], buf0, gsem0)
            pltpu.sync_copy(buf1, acc.at[cols_v.at[j1]], add=True)

        pltpu.make_async_copy(y_hbm.at[rows_v.at[HC - 1]], buf0,
                              gsem0).wait()

    plsc.subcore_barrier()
    pltpu.sync_copy(acc.at[pl.ds(base, RPT)],
                    out_hbm.at[cid, pl.ds(base, RPT)])


_hop_kernel = pl.kernel(
    _hop_body,
    out_type=jax.ShapeDtypeStruct((NC, NP, D), jnp.float32),
    mesh=_mesh,
    scratch_types=[
        pltpu.VMEM((NCHUNK // 2, CH), jnp.int32),
        pltpu.VMEM((NCHUNK // 2, CH), jnp.int32),
        pltpu.VMEM((CH, D), jnp.float32),
        pltpu.VMEM((CH, D), jnp.float32),
        pltpu.VMEM_SHARED((NP, D), jnp.float32),
        pltpu.SemaphoreType.DMA,
        pltpu.SemaphoreType.DMA,
    ],
)


def _prep_body(degp_ref, x_ref, dinv_ref, sdeg_ref, y0_ref):
    deg = 1.0 + degp_ref[0, :, 0:1] + degp_ref[1, :, 0:1]
    dv = lax.rsqrt(deg)
    dinv_ref[...] = dv
    sdeg_ref[...] = jnp.sqrt(deg)
    y0_ref[...] = x_ref[...] * dv


def _prep(degp, x):
    return pl.pallas_call(
        _prep_body,
        out_shape=[jax.ShapeDtypeStruct((NP, 1), jnp.float32),
                   jax.ShapeDtypeStruct((NP, 1), jnp.float32),
                   jax.ShapeDtypeStruct((NP, D), jnp.float32)],
    )(degp, x)


def _mid_body(a_ref, y0_ref, dinv_ref, y1_ref):
    dv = dinv_ref[...]
    y1_ref[...] = (a_ref[0] + a_ref[1] + y0_ref[...]) * (dv * dv)


def _mid(a, y0, dinv2):
    return pl.pallas_call(
        _mid_body,
        out_shape=jax.ShapeDtypeStruct((NP, D), jnp.float32),
    )(a, y0, dinv2)


_RB = 640


def _final_body(y2_ref, sdeg_ref, wgc_ref, bgc_ref, wfc_ref, bfc_ref,
                o_ref):
    h2 = y2_ref[...] * sdeg_ref[...]
    t = jnp.dot(h2, wgc_ref[...], preferred_element_type=jnp.float32)
    t = jnp.maximum(t + bgc_ref[...], 0.0)
    t2 = jnp.dot(t, wfc_ref[...], preferred_element_type=jnp.float32)
    t2 = t2 + bfc_ref[...]
    m = jnp.max(t2, axis=1, keepdims=True)
    e = jnp.exp(t2 - m)
    s = jnp.sum(e, axis=1, keepdims=True)
    o_ref[...] = t2 - m - jnp.log(s)


def _final(y2, sdeg, W_gc, b_gc, W_fc, b_fc):
    return pl.pallas_call(
        _final_body,
        grid=(NP // _RB,),
        in_specs=[
            pl.BlockSpec((_RB, D), lambda i: (i, 0)),
            pl.BlockSpec((_RB, 1), lambda i: (i, 0)),
            pl.BlockSpec((D, D), lambda i: (0, 0)),
            pl.BlockSpec((1, D), lambda i: (0, 0)),
            pl.BlockSpec((D, NCLS), lambda i: (0, 0)),
            pl.BlockSpec((1, NCLS), lambda i: (0, 0)),
        ],
        out_specs=pl.BlockSpec((_RB, NCLS), lambda i: (i, 0)),
        out_shape=jax.ShapeDtypeStruct((NP, NCLS), jnp.float32),
    )(y2, sdeg, W_gc, b_gc, W_fc, b_fc)


_PAD_IDX = np.arange(EP - E, dtype=np.int32) % (NP - N) + N


def kernel(x, edge_index, W_gc, b_gc, W_fc, b_fc):
    ei = edge_index.astype(jnp.int32)
    pad = jnp.asarray(_PAD_IDX)
    rows2 = jnp.concatenate([ei[0], pad]).reshape(NW, NCHUNK, CH)
    cols2 = jnp.concatenate([ei[1], pad]).reshape(NW, NCHUNK, CH)
    xp = jnp.pad(x, ((0, NP - N), (0, 0)))

    degp = _deg_kernel(cols2)
    dinv2, sdeg, y0 = _prep(degp, xp)

    def hop_step(_, y):
        a = _hop_kernel(y, rows2, cols2)
        return _mid(a, y, dinv2)

    y2 = lax.fori_loop(0, 2, hop_step, y0)
    out = _final(y2, sdeg, W_gc, b_gc.reshape(1, D),
                 W_fc, b_fc.reshape(1, NCLS))
    return out[:N]


# speedup vs baseline: 24.9592x; 1.3267x over previous
import jax
import jax.numpy as jnp
import numpy as np
from jax import lax
from jax.experimental import pallas as pl
from jax.experimental.pallas import tpu as pltpu
from jax.experimental.pallas import tpu_sc as plsc

N = 10000
NP = 10240
E = 320000
EP = 327680
D = 128
NCLS = 40

NC = 2
NS = 16
NW = NC * NS
EPW = EP // NW
CH = 64
NCHUNK = EPW // CH
RPT = NP // NS

_mesh = plsc.VectorSubcoreMesh(core_axis_name="c", subcore_axis_name="s",
                               num_cores=NC, num_subcores=NS)


def _deg_body(cols_hbm, out_hbm, cols_v, ones_v, dacc):
    cid = lax.axis_index("c")
    sid = lax.axis_index("s")
    wid = cid * NS + sid

    pltpu.sync_copy(cols_hbm.at[wid], cols_v)

    zero = jnp.zeros((16,), jnp.float32)
    one = jnp.ones((16,), jnp.float32)

    @pl.loop(0, CH)
    def _(i):
        for c in range(D // 16):
            ones_v[i, pl.ds(c * 16, 16)] = zero

    base = sid * RPT
    for k in range(RPT // CH):
        pltpu.sync_copy(ones_v, dacc.at[pl.ds(base + k * CH, CH)])

    @pl.loop(0, CH)
    def _(i):
        for c in range(D // 16):
            ones_v[i, pl.ds(c * 16, 16)] = one

    plsc.subcore_barrier()

    @pl.loop(0, NCHUNK)
    def _(j):
        pltpu.sync_copy(ones_v, dacc.at[cols_v.at[j]], add=True)

    plsc.subcore_barrier()
    pltpu.sync_copy(dacc.at[pl.ds(base, RPT)],
                    out_hbm.at[cid, pl.ds(base, RPT)])


_deg_kernel = pl.kernel(
    _deg_body,
    out_type=jax.ShapeDtypeStruct((NC, NP, D), jnp.float32),
    mesh=_mesh,
    scratch_types=[
        pltpu.VMEM((NCHUNK, CH), jnp.int32),
        pltpu.VMEM((CH, D), jnp.float32),
        pltpu.VMEM_SHARED((NP, D), jnp.float32),
    ],
)


SEG = NCHUNK // 4


def _hop_body(y_hbm, rows_hbm, cols_hbm, out_hbm,
              rows_v, cols_v, bufs, gsems, ssems, acc):
    cid = lax.axis_index("c")
    sid = lax.axis_index("s")
    wid = cid * NS + sid

    zero = jnp.zeros((16,), jnp.float32)

    @pl.loop(0, CH)
    def _(i):
        for c in range(D // 16):
            bufs[0][i, pl.ds(c * 16, 16)] = zero

    base = sid * RPT
    for k in range(RPT // CH):
        pltpu.sync_copy(bufs[0], acc.at[pl.ds(base + k * CH, CH)])
    plsc.subcore_barrier()

    def wait_g(b):
        pltpu.make_async_copy(y_hbm.at[rows_v.at[0]], bufs[b],
                              gsems[b]).wait()

    def issue_g(c, b):
        pltpu.async_copy(y_hbm.at[rows_v.at[c]], bufs[b], gsems[b])

    def wait_s(b):
        pltpu.make_async_copy(bufs[b], acc.at[cols_v.at[0]],
                              ssems[b]).wait()

    def issue_s(c, b):
        pltpu.async_copy(bufs[b], acc.at[cols_v.at[c]], ssems[b],
                         add=True)

    for seg in range(NCHUNK // SEG):
        pltpu.sync_copy(rows_hbm.at[wid, pl.ds(seg * SEG, SEG)], rows_v)
        pltpu.sync_copy(cols_hbm.at[wid, pl.ds(seg * SEG, SEG)], cols_v)

        issue_g(0, 0)
        issue_g(1, 1)
        wait_g(0)
        issue_s(0, 0)
        issue_g(2, 2)
        wait_g(1)
        issue_s(1, 1)

        @pl.loop(0, (SEG - 4) // 3)
        def _(g):
            c0 = 3 * g + 2
            for i in range(3):
                c = c0 + i
                b = (2 + i) % 3
                wait_s((b + 1) % 3)
                issue_g(c + 1, (b + 1) % 3)
                wait_g(b)
                issue_s(c, b)

        wait_s(0)
        issue_g(SEG - 1, 0)
        wait_g(2)
        issue_s(SEG - 2, 2)
        wait_s(1)
        wait_g(0)
        issue_s(SEG - 1, 0)
        wait_s(2)
        wait_s(0)

    plsc.subcore_barrier()
    pltpu.sync_copy(acc.at[pl.ds(base, RPT)],
                    out_hbm.at[cid, pl.ds(base, RPT)])


_hop_kernel = pl.kernel(
    _hop_body,
    out_type=jax.ShapeDtypeStruct((NC, NP, D), jnp.float32),
    mesh=_mesh,
    scratch_types=[
        pltpu.VMEM((SEG, CH), jnp.int32),
        pltpu.VMEM((SEG, CH), jnp.int32),
        [pltpu.VMEM((CH, D), jnp.float32) for _ in range(3)],
        [pltpu.SemaphoreType.DMA for _ in range(3)],
        [pltpu.SemaphoreType.DMA for _ in range(3)],
        pltpu.VMEM_SHARED((NP, D), jnp.float32),
    ],
)


def _prep_body(degp_ref, x_ref, dinv_ref, sdeg_ref, y0_ref):
    deg = 1.0 + degp_ref[0, :, 0:1] + degp_ref[1, :, 0:1]
    dv = lax.rsqrt(deg)
    dinv_ref[...] = dv
    sdeg_ref[...] = jnp.sqrt(deg)
    y0_ref[...] = x_ref[...] * dv


def _prep(degp, x):
    return pl.pallas_call(
        _prep_body,
        out_shape=[jax.ShapeDtypeStruct((NP, 1), jnp.float32),
                   jax.ShapeDtypeStruct((NP, 1), jnp.float32),
                   jax.ShapeDtypeStruct((NP, D), jnp.float32)],
    )(degp, x)


def _mid_body(a_ref, y0_ref, dinv_ref, y1_ref):
    dv = dinv_ref[...]
    y1_ref[...] = (a_ref[0] + a_ref[1] + y0_ref[...]) * (dv * dv)


def _mid(a, y0, dinv2):
    return pl.pallas_call(
        _mid_body,
        out_shape=jax.ShapeDtypeStruct((NP, D), jnp.float32),
    )(a, y0, dinv2)


_RB = 640


def _final_body(y2_ref, sdeg_ref, wgc_ref, bgc_ref, wfc_ref, bfc_ref,
                o_ref):
    h2 = y2_ref[...] * sdeg_ref[...]
    t = jnp.dot(h2, wgc_ref[...], preferred_element_type=jnp.float32)
    t = jnp.maximum(t + bgc_ref[...], 0.0)
    t2 = jnp.dot(t, wfc_ref[...], preferred_element_type=jnp.float32)
    t2 = t2 + bfc_ref[...]
    m = jnp.max(t2, axis=1, keepdims=True)
    e = jnp.exp(t2 - m)
    s = jnp.sum(e, axis=1, keepdims=True)
    o_ref[...] = t2 - m - jnp.log(s)


def _final(y2, sdeg, W_gc, b_gc, W_fc, b_fc):
    return pl.pallas_call(
        _final_body,
        grid=(NP // _RB,),
        in_specs=[
            pl.BlockSpec((_RB, D), lambda i: (i, 0)),
            pl.BlockSpec((_RB, 1), lambda i: (i, 0)),
            pl.BlockSpec((D, D), lambda i: (0, 0)),
            pl.BlockSpec((1, D), lambda i: (0, 0)),
            pl.BlockSpec((D, NCLS), lambda i: (0, 0)),
            pl.BlockSpec((1, NCLS), lambda i: (0, 0)),
        ],
        out_specs=pl.BlockSpec((_RB, NCLS), lambda i: (i, 0)),
        out_shape=jax.ShapeDtypeStruct((NP, NCLS), jnp.float32),
    )(y2, sdeg, W_gc, b_gc, W_fc, b_fc)


_PAD_IDX = np.arange(EP - E, dtype=np.int32) % (NP - N) + N


def kernel(x, edge_index, W_gc, b_gc, W_fc, b_fc):
    ei = edge_index.astype(jnp.int32)
    pad = jnp.asarray(_PAD_IDX)
    rows2 = jnp.concatenate([ei[0], pad]).reshape(NW, NCHUNK, CH)
    cols2 = jnp.concatenate([ei[1], pad]).reshape(NW, NCHUNK, CH)
    xp = jnp.pad(x, ((0, NP - N), (0, 0)))

    degp = _deg_kernel(cols2)
    dinv2, sdeg, y0 = _prep(degp, xp)

    def hop_step(_, y):
        a = _hop_kernel(y, rows2, cols2)
        return _mid(a, y, dinv2)

    y2 = lax.fori_loop(0, 2, hop_step, y0)
    out = _final(y2, sdeg, W_gc, b_gc.reshape(1, D),
                 W_fc, b_fc.reshape(1, NCLS))
    return out[:N]


# speedup vs baseline: 27.7724x; 1.1127x over previous
import jax
import jax.numpy as jnp
import numpy as np
from jax import lax
from jax.experimental import pallas as pl
from jax.experimental.pallas import tpu as pltpu
from jax.experimental.pallas import tpu_sc as plsc

N = 10000
NP = 10240
E = 320000
EP = 327680
D = 128
NCLS = 40

NC = 2
NS = 16
NW = NC * NS
EPW = EP // NW
CH = 64
NCHUNK = EPW // CH
RPT = NP // NS

_mesh = plsc.VectorSubcoreMesh(core_axis_name="c", subcore_axis_name="s",
                               num_cores=NC, num_subcores=NS)

SEG = NCHUNK // 4

import dataclasses as _dataclasses
_cp = pltpu.CompilerParams()
if "needs_layout_passes" in pltpu.CompilerParams.__dataclass_fields__:
    _cp = _dataclasses.replace(_cp, needs_layout_passes=False)


_HR = NP // D


def _deg_body(cols_hbm, out_hbm, cols_v, histo, acc8, tmp8, stg):
    cid = lax.axis_index("c")
    sid = lax.axis_index("s")
    wid = cid * NS + sid

    zero = jnp.zeros((16,), jnp.float32)
    one = jnp.ones((16,), jnp.float32)

    @pl.loop(0, _HR)
    def _(i):
        for c in range(D // 16):
            histo[i, pl.ds(c * 16, 16)] = zero

    for seg in range(NCHUNK // SEG):
        pltpu.sync_copy(cols_hbm.at[wid, pl.ds(seg * SEG, SEG)], cols_v)

        @pl.loop(0, SEG)
        def _(j):
            for k in range(CH // 16):
                idx = cols_v[j, pl.ds(k * 16, 16)]
                r = lax.shift_right_logical(idx, 7)
                l = lax.bitwise_and(idx, 127)
                plsc.addupdate_scatter(histo, [r, l], one)

    pltpu.sync_copy(histo, stg.at[sid])
    plsc.subcore_barrier()

    @pl.when(sid < _HR // 8)
    def _():
        rb = sid * 8
        pltpu.sync_copy(stg.at[0, pl.ds(rb, 8)], acc8)
        for k in range(1, NS):
            pltpu.sync_copy(stg.at[k, pl.ds(rb, 8)], tmp8)

            @pl.loop(0, 8)
            def _(i):
                for c in range(D // 16):
                    s = pl.ds(c * 16, 16)
                    acc8[i, s] = acc8[i, s] + tmp8[i, s]

        pltpu.sync_copy(acc8, out_hbm.at[cid, pl.ds(rb, 8)])


_deg_kernel = pl.kernel(
    _deg_body,
    out_type=jax.ShapeDtypeStruct((NC, _HR, D), jnp.float32),
    mesh=_mesh,
    scratch_types=[
        pltpu.VMEM((SEG, CH), jnp.int32),
        pltpu.VMEM((_HR, D), jnp.float32),
        pltpu.VMEM((8, D), jnp.float32),
        pltpu.VMEM((8, D), jnp.float32),
        pltpu.VMEM_SHARED((NS, _HR, D), jnp.float32),
    ],
    compiler_params=_cp,
)


def _hop_body(y_hbm, rows_hbm, cols_hbm, out_hbm,
              rows_v, cols_v, bufs, gsems, ssems, acc):
    cid = lax.axis_index("c")
    sid = lax.axis_index("s")
    wid = cid * NS + sid

    zero = jnp.zeros((16,), jnp.float32)

    @pl.loop(0, CH)
    def _(i):
        for c in range(D // 16):
            bufs[0][i, pl.ds(c * 16, 16)] = zero

    base = sid * RPT
    for k in range(RPT // CH):
        pltpu.sync_copy(bufs[0], acc.at[pl.ds(base + k * CH, CH)])
    plsc.subcore_barrier()

    def wait_g(b):
        pltpu.make_async_copy(y_hbm.at[rows_v.at[0]], bufs[b],
                              gsems[b]).wait()

    def issue_g(c, b):
        pltpu.async_copy(y_hbm.at[rows_v.at[c]], bufs[b], gsems[b])

    def wait_s(b):
        pltpu.make_async_copy(bufs[b], acc.at[cols_v.at[0]],
                              ssems[b]).wait()

    def issue_s(c, b):
        pltpu.async_copy(bufs[b], acc.at[cols_v.at[c]], ssems[b],
                         add=True)

    for seg in range(NCHUNK // SEG):
        pltpu.sync_copy(rows_hbm.at[wid, pl.ds(seg * SEG, SEG)], rows_v)
        pltpu.sync_copy(cols_hbm.at[wid, pl.ds(seg * SEG, SEG)], cols_v)

        issue_g(0, 0)
        issue_g(1, 1)
        wait_g(0)
        issue_s(0, 0)
        issue_g(2, 2)
        wait_g(1)
        issue_s(1, 1)

        @pl.loop(0, (SEG - 4) // 3)
        def _(g):
            c0 = 3 * g + 2
            for i in range(3):
                c = c0 + i
                b = (2 + i) % 3
                wait_s((b + 1) % 3)
                issue_g(c + 1, (b + 1) % 3)
                wait_g(b)
                issue_s(c, b)

        wait_s(0)
        issue_g(SEG - 1, 0)
        wait_g(2)
        issue_s(SEG - 2, 2)
        wait_s(1)
        wait_g(0)
        issue_s(SEG - 1, 0)
        wait_s(2)
        wait_s(0)

    plsc.subcore_barrier()
    pltpu.sync_copy(acc.at[pl.ds(base, RPT)],
                    out_hbm.at[cid, pl.ds(base, RPT)])


_hop_kernel = pl.kernel(
    _hop_body,
    out_type=jax.ShapeDtypeStruct((NC, NP, D), jnp.float32),
    mesh=_mesh,
    scratch_types=[
        pltpu.VMEM((SEG, CH), jnp.int32),
        pltpu.VMEM((SEG, CH), jnp.int32),
        [pltpu.VMEM((CH, D), jnp.float32) for _ in range(3)],
        [pltpu.SemaphoreType.DMA for _ in range(3)],
        [pltpu.SemaphoreType.DMA for _ in range(3)],
        pltpu.VMEM_SHARED((NP, D), jnp.float32),
    ],
)


def _prep_body(degp_ref, x_ref, dinv_ref, sdeg_ref, y0_ref):
    deg = 1.0 + degp_ref[0] + degp_ref[1]
    dv = lax.rsqrt(deg)
    dinv_ref[...] = dv
    sdeg_ref[...] = jnp.sqrt(deg)
    y0_ref[...] = x_ref[...] * dv


def _prep(degp, x):
    return pl.pallas_call(
        _prep_body,
        out_shape=[jax.ShapeDtypeStruct((NP, 1), jnp.float32),
                   jax.ShapeDtypeStruct((NP, 1), jnp.float32),
                   jax.ShapeDtypeStruct((NP, D), jnp.float32)],
    )(degp, x)


def _mid_body(a_ref, y0_ref, dinv_ref, y1_ref):
    dv = dinv_ref[...]
    y1_ref[...] = (a_ref[0] + a_ref[1] + y0_ref[...]) * (dv * dv)


def _mid(a, y0, dinv2):
    return pl.pallas_call(
        _mid_body,
        out_shape=jax.ShapeDtypeStruct((NP, D), jnp.float32),
    )(a, y0, dinv2)


_RB = 640


def _final_body(y2_ref, sdeg_ref, wgc_ref, bgc_ref, wfc_ref, bfc_ref,
                o_ref):
    h2 = y2_ref[...] * sdeg_ref[...]
    t = jnp.dot(h2, wgc_ref[...], preferred_element_type=jnp.float32)
    t = jnp.maximum(t + bgc_ref[...], 0.0)
    t2 = jnp.dot(t, wfc_ref[...], preferred_element_type=jnp.float32)
    t2 = t2 + bfc_ref[...]
    m = jnp.max(t2, axis=1, keepdims=True)
    e = jnp.exp(t2 - m)
    s = jnp.sum(e, axis=1, keepdims=True)
    o_ref[...] = t2 - m - jnp.log(s)


def _final(y2, sdeg, W_gc, b_gc, W_fc, b_fc):
    return pl.pallas_call(
        _final_body,
        grid=(NP // _RB,),
        in_specs=[
            pl.BlockSpec((_RB, D), lambda i: (i, 0)),
            pl.BlockSpec((_RB, 1), lambda i: (i, 0)),
            pl.BlockSpec((D, D), lambda i: (0, 0)),
            pl.BlockSpec((1, D), lambda i: (0, 0)),
            pl.BlockSpec((D, NCLS), lambda i: (0, 0)),
            pl.BlockSpec((1, NCLS), lambda i: (0, 0)),
        ],
        out_specs=pl.BlockSpec((_RB, NCLS), lambda i: (i, 0)),
        out_shape=jax.ShapeDtypeStruct((NP, NCLS), jnp.float32),
    )(y2, sdeg, W_gc, b_gc, W_fc, b_fc)


_PAD_IDX = np.arange(EP - E, dtype=np.int32) % (NP - N) + N


def kernel(x, edge_index, W_gc, b_gc, W_fc, b_fc):
    ei = edge_index.astype(jnp.int32)
    pad = jnp.asarray(_PAD_IDX)
    rows2 = jnp.concatenate([ei[0], pad]).reshape(NW, NCHUNK, CH)
    cols2 = jnp.concatenate([ei[1], pad]).reshape(NW, NCHUNK, CH)
    xp = jnp.pad(x, ((0, NP - N), (0, 0)))

    degp = _deg_kernel(cols2)
    deglin = degp.reshape(NC, NP, 1)
    dinv2, sdeg, y0 = _prep(deglin, xp)

    def hop_step(_, y):
        a = _hop_kernel(y, rows2, cols2)
        return _mid(a, y, dinv2)

    y2 = lax.fori_loop(0, 2, hop_step, y0)
    out = _final(y2, sdeg, W_gc, b_gc.reshape(1, D),
                 W_fc, b_fc.reshape(1, NCLS))
    return out[:N]


# speedup vs baseline: 28.6586x; 1.0319x over previous
import jax
import jax.numpy as jnp
import numpy as np
from jax import lax
from jax.experimental import pallas as pl
from jax.experimental.pallas import tpu as pltpu
from jax.experimental.pallas import tpu_sc as plsc

N = 10000
NP = 10240
E = 320000
EP = 327680
D = 128
NCLS = 40

NC = 2
NS = 16
NW = NC * NS
EPW = EP // NW
CH = 64
NCHUNK = EPW // CH
RPT = NP // NS

_mesh = plsc.VectorSubcoreMesh(core_axis_name="c", subcore_axis_name="s",
                               num_cores=NC, num_subcores=NS)

SEG = NCHUNK // 4

import dataclasses as _dataclasses
_cp = pltpu.CompilerParams()
if "needs_layout_passes" in pltpu.CompilerParams.__dataclass_fields__:
    _cp = _dataclasses.replace(_cp, needs_layout_passes=False)


_HR = NP // D


def _deg_body(cols_hbm, out_hbm, cols_v, histo, acc8, tmp8, stg):
    cid = lax.axis_index("c")
    sid = lax.axis_index("s")
    wid = cid * NS + sid

    zero = jnp.zeros((16,), jnp.float32)
    one = jnp.ones((16,), jnp.float32)

    @pl.loop(0, _HR)
    def _(i):
        for c in range(D // 16):
            histo[i, pl.ds(c * 16, 16)] = zero

    for seg in range(NCHUNK // SEG):
        pltpu.sync_copy(cols_hbm.at[wid, pl.ds(seg * SEG, SEG)], cols_v)

        @pl.loop(0, SEG)
        def _(j):
            for k in range(CH // 16):
                idx = cols_v[j, pl.ds(k * 16, 16)]
                r = lax.shift_right_logical(idx, 7)
                l = lax.bitwise_and(idx, 127)
                plsc.addupdate_scatter(histo, [r, l], one)

    pltpu.sync_copy(histo, stg.at[sid])
    plsc.subcore_barrier()

    @pl.when(sid < _HR // 8)
    def _():
        rb = sid * 8
        pltpu.sync_copy(stg.at[0, pl.ds(rb, 8)], acc8)
        for k in range(1, NS):
            pltpu.sync_copy(stg.at[k, pl.ds(rb, 8)], tmp8)

            @pl.loop(0, 8)
            def _(i):
                for c in range(D // 16):
                    s = pl.ds(c * 16, 16)
                    acc8[i, s] = acc8[i, s] + tmp8[i, s]

        pltpu.sync_copy(acc8, out_hbm.at[cid, pl.ds(rb, 8)])


_deg_kernel = pl.kernel(
    _deg_body,
    out_type=jax.ShapeDtypeStruct((NC, _HR, D), jnp.float32),
    mesh=_mesh,
    scratch_types=[
        pltpu.VMEM((SEG, CH), jnp.int32),
        pltpu.VMEM((_HR, D), jnp.float32),
        pltpu.VMEM((8, D), jnp.float32),
        pltpu.VMEM((8, D), jnp.float32),
        pltpu.VMEM_SHARED((NS, _HR, D), jnp.float32),
    ],
    compiler_params=_cp,
)


def _hop_body(y_hbm, rows_hbm, cols_hbm, out_hbm,
              rows_v, cols_v, bufs, gsems, ssems, isem, acc):
    cid = lax.axis_index("c")
    sid = lax.axis_index("s")
    wid = cid * NS + sid

    zero = jnp.zeros((16,), jnp.float32)

    def idx_issue(seg):
        b = seg % 2
        pltpu.async_copy(rows_hbm.at[wid, pl.ds(seg * SEG, SEG)],
                         rows_v[b], isem)
        pltpu.async_copy(cols_hbm.at[wid, pl.ds(seg * SEG, SEG)],
                         cols_v[b], isem)

    def idx_wait(seg):
        b = seg % 2
        pltpu.make_async_copy(rows_hbm.at[wid, pl.ds(seg * SEG, SEG)],
                              rows_v[b], isem).wait()
        pltpu.make_async_copy(cols_hbm.at[wid, pl.ds(seg * SEG, SEG)],
                              cols_v[b], isem).wait()

    idx_issue(0)

    @pl.loop(0, CH)
    def _(i):
        for c in range(D // 16):
            bufs[0][i, pl.ds(c * 16, 16)] = zero

    base = sid * RPT
    for k in range(RPT // CH):
        pltpu.async_copy(bufs[0], acc.at[pl.ds(base + k * CH, CH)],
                         gsems[0])
    for k in range(RPT // CH):
        pltpu.make_async_copy(bufs[0], acc.at[pl.ds(base, CH)],
                              gsems[0]).wait()
    plsc.subcore_barrier()

    for seg in range(NCHUNK // SEG):
        rv, cv = rows_v[seg % 2], cols_v[seg % 2]

        def wait_g(b):
            pltpu.make_async_copy(y_hbm.at[rv.at[0]], bufs[b],
                                  gsems[b]).wait()

        def issue_g(c, b):
            pltpu.async_copy(y_hbm.at[rv.at[c]], bufs[b], gsems[b])

        def wait_s(b):
            pltpu.make_async_copy(bufs[b], acc.at[cv.at[0]],
                                  ssems[b]).wait()

        def issue_s(c, b):
            pltpu.async_copy(bufs[b], acc.at[cv.at[c]], ssems[b],
                             add=True)

        idx_wait(seg)
        issue_g(0, 0)
        issue_g(1, 1)
        if seg + 1 < NCHUNK // SEG:
            idx_issue(seg + 1)
        wait_g(0)
        issue_s(0, 0)
        issue_g(2, 2)
        wait_g(1)
        issue_s(1, 1)

        @pl.loop(0, (SEG - 4) // 3)
        def _(g):
            c0 = 3 * g + 2
            for i in range(3):
                c = c0 + i
                b = (2 + i) % 3
                wait_s((b + 1) % 3)
                issue_g(c + 1, (b + 1) % 3)
                wait_g(b)
                issue_s(c, b)

        wait_s(0)
        issue_g(SEG - 1, 0)
        wait_g(2)
        issue_s(SEG - 2, 2)
        wait_s(1)
        wait_g(0)
        issue_s(SEG - 1, 0)
        wait_s(2)
        wait_s(0)

    plsc.subcore_barrier()
    pltpu.sync_copy(acc.at[pl.ds(base, RPT)],
                    out_hbm.at[cid, pl.ds(base, RPT)])


_hop_kernel = pl.kernel(
    _hop_body,
    out_type=jax.ShapeDtypeStruct((NC, NP, D), jnp.float32),
    mesh=_mesh,
    scratch_types=[
        [pltpu.VMEM((SEG, CH), jnp.int32) for _ in range(2)],
        [pltpu.VMEM((SEG, CH), jnp.int32) for _ in range(2)],
        [pltpu.VMEM((CH, D), jnp.float32) for _ in range(3)],
        [pltpu.SemaphoreType.DMA for _ in range(3)],
        [pltpu.SemaphoreType.DMA for _ in range(3)],
        pltpu.SemaphoreType.DMA,
        pltpu.VMEM_SHARED((NP, D), jnp.float32),
    ],
)


def _prep_body(degp_ref, x_ref, dinv_ref, sdeg_ref, y0_ref):
    deg = 1.0 + degp_ref[0] + degp_ref[1]
    dv = lax.rsqrt(deg)
    dinv_ref[...] = dv
    sdeg_ref[...] = jnp.sqrt(deg)
    y0_ref[...] = x_ref[...] * dv


def _prep(degp, x):
    return pl.pallas_call(
        _prep_body,
        out_shape=[jax.ShapeDtypeStruct((NP, 1), jnp.float32),
                   jax.ShapeDtypeStruct((NP, 1), jnp.float32),
                   jax.ShapeDtypeStruct((NP, D), jnp.float32)],
    )(degp, x)


def _mid_body(a_ref, y0_ref, dinv_ref, y1_ref):
    dv = dinv_ref[...]
    y1_ref[...] = (a_ref[0] + a_ref[1] + y0_ref[...]) * (dv * dv)


def _mid(a, y0, dinv2):
    return pl.pallas_call(
        _mid_body,
        out_shape=jax.ShapeDtypeStruct((NP, D), jnp.float32),
    )(a, y0, dinv2)


_RB = 640


def _final_body(y2_ref, sdeg_ref, wgc_ref, bgc_ref, wfc_ref, bfc_ref,
                o_ref):
    h2 = y2_ref[...] * sdeg_ref[...]
    t = jnp.dot(h2, wgc_ref[...], preferred_element_type=jnp.float32)
    t = jnp.maximum(t + bgc_ref[...], 0.0)
    t2 = jnp.dot(t, wfc_ref[...], preferred_element_type=jnp.float32)
    t2 = t2 + bfc_ref[...]
    m = jnp.max(t2, axis=1, keepdims=True)
    e = jnp.exp(t2 - m)
    s = jnp.sum(e, axis=1, keepdims=True)
    o_ref[...] = t2 - m - jnp.log(s)


def _final(y2, sdeg, W_gc, b_gc, W_fc, b_fc):
    return pl.pallas_call(
        _final_body,
        grid=(NP // _RB,),
        in_specs=[
            pl.BlockSpec((_RB, D), lambda i: (i, 0)),
            pl.BlockSpec((_RB, 1), lambda i: (i, 0)),
            pl.BlockSpec((D, D), lambda i: (0, 0)),
            pl.BlockSpec((1, D), lambda i: (0, 0)),
            pl.BlockSpec((D, NCLS), lambda i: (0, 0)),
            pl.BlockSpec((1, NCLS), lambda i: (0, 0)),
        ],
        out_specs=pl.BlockSpec((_RB, NCLS), lambda i: (i, 0)),
        out_shape=jax.ShapeDtypeStruct((NP, NCLS), jnp.float32),
    )(y2, sdeg, W_gc, b_gc, W_fc, b_fc)


_PAD_IDX = np.arange(EP - E, dtype=np.int32) % (NP - N) + N


def kernel(x, edge_index, W_gc, b_gc, W_fc, b_fc):
    ei = edge_index.astype(jnp.int32)
    pad = jnp.asarray(_PAD_IDX)
    rows2 = jnp.concatenate([ei[0], pad]).reshape(NW, NCHUNK, CH)
    cols2 = jnp.concatenate([ei[1], pad]).reshape(NW, NCHUNK, CH)
    xp = jnp.pad(x, ((0, NP - N), (0, 0)))

    degp = _deg_kernel(cols2)
    deglin = degp.reshape(NC, NP, 1)
    dinv2, sdeg, y0 = _prep(deglin, xp)

    def hop_step(_, y):
        a = _hop_kernel(y, rows2, cols2)
        return _mid(a, y, dinv2)

    y2 = lax.fori_loop(0, 2, hop_step, y0)
    out = _final(y2, sdeg, W_gc, b_gc.reshape(1, D),
                 W_fc, b_fc.reshape(1, NCLS))
    return out[:N]
